# Initial kernel scaffold; baseline (speedup 1.0000x reference)
import jax
import jax.numpy as jnp
from jax.experimental import pallas as pl


def kernel(x, edge_index, edge_attr, nn1_w1, nn1_b1, nn1_w2, nn1_b2, root1, bias1, bn1_g, bn1_b, nn2_w1, nn2_b1, nn2_w2, nn2_b2, root2, bias2, bn2_g, bn2_b, lin1_w, lin1_b, lin2_w, lin2_b):
    raise NotImplementedError("write your pallas kernel here")


# speedup vs baseline: 1.4580x; 1.4580x over previous
import functools
import math

import jax
import jax.numpy as jnp
from jax import lax
from jax.experimental import pallas as pl
from jax.experimental.pallas import tpu as pltpu
from jax.experimental.pallas import tpu_sc as plsc

_N = 10000
_E = 160000
_F = 16
_OUT = 8

_NC = 2
_NS = 16
_NW = _NC * _NS
_EPW = _E // _NW
_CH = 125
_NCH = _EPW // _CH
_IDXROWS = _E // _CH

_NPAD = 10112
_RPT = _NPAD // _NS

_BE = 3200
_GRID = _E // _BE

_MESH = plsc.VectorSubcoreMesh(
    core_axis_name="c", subcore_axis_name="s", num_cores=_NC, num_subcores=_NS
)
_SC_PARAMS = pltpu.CompilerParams(use_tc_tiling_on_sc=False)


def _sc_gather_count(x, src2d, dst2d, ones_blk, zeros_blk):
    @functools.partial(
        pl.kernel,
        out_type=[
            jax.ShapeDtypeStruct((_E, _F), jnp.float32),
            jax.ShapeDtypeStruct((_NC, _NPAD, _F), jnp.float32),
        ],
        mesh=_MESH,
        compiler_params=_SC_PARAMS,
        scratch_types=[
            pltpu.VMEM((_NCH, _CH), jnp.int32),
            pltpu.VMEM((_NCH, _CH), jnp.int32),
            pltpu.VMEM((_EPW, _F), jnp.float32),
            pltpu.VMEM((_CH, _F), jnp.float32),
            pltpu.SemaphoreType.DMA,
            pltpu.SemaphoreType.DMA,
            pltpu.VMEM_SHARED((_NPAD, _F), jnp.float32),
        ],
    )
    def k(x_hbm, src_hbm, dst_hbm, ones_hbm, zeros_hbm, xs_out, cnt_out,
          src_v, dst_v, rows_v, ones_v, gsem, ssem, acc):
        c = lax.axis_index("c")
        s = lax.axis_index("s")
        wid = c * _NS + s
        base = wid * _EPW
        crow = wid * _NCH
        pltpu.sync_copy(src_hbm.at[pl.ds(crow, _NCH)], src_v)
        pltpu.sync_copy(dst_hbm.at[pl.ds(crow, _NCH)], dst_v)
        pltpu.sync_copy(ones_hbm, ones_v)
        pltpu.sync_copy(zeros_hbm, acc.at[pl.ds(s * _RPT, _RPT)])
        gds = []
        for j in range(_NCH):
            gds.append(pltpu.async_copy(
                x_hbm.at[src_v.at[j]], rows_v.at[pl.ds(j * _CH, _CH)], gsem))
        plsc.subcore_barrier()
        sds = []
        for j in range(_NCH):
            sds.append(pltpu.async_copy(
                ones_v, acc.at[dst_v.at[j]], ssem, add=True))
        for d in gds:
            d.wait()
        pltpu.sync_copy(rows_v, xs_out.at[pl.ds(base, _EPW)])
        for d in sds:
            d.wait()
        plsc.subcore_barrier()
        pltpu.sync_copy(acc.at[pl.ds(s * _RPT, _RPT)],
                        cnt_out.at[c, pl.ds(s * _RPT, _RPT)])

    return k(x, src2d, dst2d, ones_blk, zeros_blk)


def _sc_scatter(msg, dst2d, zeros_blk):
    @functools.partial(
        pl.kernel,
        out_type=jax.ShapeDtypeStruct((_NC, _NPAD, _F), jnp.float32),
        mesh=_MESH,
        compiler_params=_SC_PARAMS,
        scratch_types=[
            pltpu.VMEM((_NCH, _CH), jnp.int32),
            pltpu.VMEM((_EPW, _F), jnp.float32),
            pltpu.SemaphoreType.DMA,
            pltpu.VMEM_SHARED((_NPAD, _F), jnp.float32),
        ],
    )
    def k(msg_hbm, dst_hbm, zeros_hbm, part_out, dst_v, rows_v, ssem, acc):
        c = lax.axis_index("c")
        s = lax.axis_index("s")
        wid = c * _NS + s
        base = wid * _EPW
        crow = wid * _NCH
        pltpu.sync_copy(dst_hbm.at[pl.ds(crow, _NCH)], dst_v)
        pltpu.sync_copy(msg_hbm.at[pl.ds(base, _EPW)], rows_v)
        pltpu.sync_copy(zeros_hbm, acc.at[pl.ds(s * _RPT, _RPT)])
        plsc.subcore_barrier()
        sds = []
        for j in range(_NCH):
            sds.append(pltpu.async_copy(
                rows_v.at[pl.ds(j * _CH, _CH)], acc.at[dst_v.at[j]], ssem,
                add=True))
        for d in sds:
            d.wait()
        plsc.subcore_barrier()
        pltpu.sync_copy(acc.at[pl.ds(s * _RPT, _RPT)],
                        part_out.at[c, pl.ds(s * _RPT, _RPT)])

    return k(msg, dst2d, zeros_blk)


def _sc_gather(table, idx2d, width):
    @functools.partial(
        pl.kernel,
        out_type=jax.ShapeDtypeStruct((_E, width), jnp.float32),
        mesh=_MESH,
        compiler_params=_SC_PARAMS,
        scratch_types=[
            pltpu.VMEM((_NCH, _CH), jnp.int32),
            pltpu.VMEM((_EPW, width), jnp.float32),
            pltpu.SemaphoreType.DMA,
        ],
    )
    def k(tab_hbm, idx_hbm, out_hbm, idx_v, rows_v, gsem):
        c = lax.axis_index("c")
        s = lax.axis_index("s")
        wid = c * _NS + s
        base = wid * _EPW
        crow = wid * _NCH
        pltpu.sync_copy(idx_hbm.at[pl.ds(crow, _NCH)], idx_v)
        gds = []
        for j in range(_NCH):
            gds.append(pltpu.async_copy(
                tab_hbm.at[idx_v.at[j]], rows_v.at[pl.ds(j * _CH, _CH)], gsem))
        for d in gds:
            d.wait()
        pltpu.sync_copy(rows_v, out_hbm.at[pl.ds(base, _EPW)])

    return k(table, idx2d)


def _tc_msg(ea, xs, w1t, b1r, w2t, b2r):
    def body(ea_ref, xs_ref, w1t_ref, b1_ref, w2t_ref, b2_ref, out_ref):
        he = jnp.maximum(
            jnp.dot(ea_ref[...], w1t_ref[...],
                    preferred_element_type=jnp.float32) + b1_ref[...], 0.0)
        wflat = jnp.dot(he, w2t_ref[...],
                        preferred_element_type=jnp.float32) + b2_ref[...]
        xsb = xs_ref[...]
        acc = xsb[:, 0:1] * wflat[:, 0:_F]
        for i in range(1, _F):
            acc = acc + xsb[:, i:i + 1] * wflat[:, i * _F:(i + 1) * _F]
        out_ref[...] = acc

    return pl.pallas_call(
        body,
        grid=(_GRID,),
        in_specs=[
            pl.BlockSpec((_BE, _F), lambda i: (i, 0)),
            pl.BlockSpec((_BE, _F), lambda i: (i, 0)),
            pl.BlockSpec((_F, _F), lambda i: (0, 0)),
            pl.BlockSpec((1, _F), lambda i: (0, 0)),
            pl.BlockSpec((_F, _F * _F), lambda i: (0, 0)),
            pl.BlockSpec((1, _F * _F), lambda i: (0, 0)),
        ],
        out_specs=pl.BlockSpec((_BE, _F), lambda i: (i, 0)),
        out_shape=jax.ShapeDtypeStruct((_E, _F), jnp.float32),
    )(ea, xs, w1t, b1r, w2t, b2r)


def _tc_combine(sp, cp, x, root1, bi1, g1, bb1, root2, bi2):
    def body(sp_ref, cp_ref, x_ref, r1_ref, bi1_ref, g_ref, bb_ref,
             r2_ref, bi2_ref, h1_ref, xr2_ref):
        ssum = sp_ref[0, :_N, :] + sp_ref[1, :_N, :]
        csum = cp_ref[0, :_N, :] + cp_ref[1, :_N, :]
        agg = ssum / jnp.maximum(csum[:, 0:1], 1.0)
        pre = agg + jnp.dot(x_ref[...], r1_ref[...],
                            preferred_element_type=jnp.float32) + bi1_ref[...]
        h1 = jnp.maximum(pre * g_ref[...] + bb_ref[...], 0.0)
        h1_ref[...] = h1
        xr2_ref[...] = jnp.dot(h1, r2_ref[...],
                               preferred_element_type=jnp.float32) + bi2_ref[...]

    return pl.pallas_call(
        body,
        out_shape=[
            jax.ShapeDtypeStruct((_N, _F), jnp.float32),
            jax.ShapeDtypeStruct((_N, _F), jnp.float32),
        ],
    )(sp, cp, x, root1, bi1, g1, bb1, root2, bi2)


def _tc_final(sp2, cp, xr2, g2, bb2, l1t, l1b, l2t, l2b):
    def body(sp_ref, cp_ref, xr2_ref, g_ref, bb_ref, l1t_ref, l1b_ref,
             l2t_ref, l2b_ref, out_ref):
        ssum = sp_ref[0, :_N, :] + sp_ref[1, :_N, :]
        csum = cp_ref[0, :_N, :] + cp_ref[1, :_N, :]
        agg = ssum / jnp.maximum(csum[:, 0:1], 1.0)
        pre = agg + xr2_ref[...]
        h2 = jnp.maximum(pre * g_ref[...] + bb_ref[...], 0.0)
        h3 = jnp.maximum(
            jnp.dot(h2, l1t_ref[...],
                    preferred_element_type=jnp.float32) + l1b_ref[...], 0.0)
        out_ref[...] = jnp.dot(h3, l2t_ref[...],
                               preferred_element_type=jnp.float32) + l2b_ref[...]

    return pl.pallas_call(
        body,
        out_shape=jax.ShapeDtypeStruct((_N, _OUT), jnp.float32),
    )(sp2, cp, xr2, g2, bb2, l1t, l1b, l2t, l2b)


def kernel(x, edge_index, edge_attr, nn1_w1, nn1_b1, nn1_w2, nn1_b2, root1,
           bias1, bn1_g, bn1_b, nn2_w1, nn2_b1, nn2_w2, nn2_b2, root2, bias2,
           bn2_g, bn2_b, lin1_w, lin1_b, lin2_w, lin2_b):
    src2d = edge_index[0].reshape(_IDXROWS, _CH)
    dst2d = edge_index[1].reshape(_IDXROWS, _CH)
    ones_blk = jnp.ones((_CH, _F), jnp.float32)
    zeros_blk = jnp.zeros((_RPT, _F), jnp.float32)

    bn_scale = 1.0 / math.sqrt(1.0 + 1e-5)
    g1 = (bn1_g * bn_scale).reshape(1, _F)
    bb1 = bn1_b.reshape(1, _F)
    g2 = (bn2_g * bn_scale).reshape(1, _F)
    bb2 = bn2_b.reshape(1, _F)

    xs, cntp = _sc_gather_count(x, src2d, dst2d, ones_blk, zeros_blk)
    msg1 = _tc_msg(edge_attr, xs, nn1_w1.T, nn1_b1.reshape(1, _F),
                   nn1_w2.T, nn1_b2.reshape(1, _F * _F))
    sp1 = _sc_scatter(msg1, dst2d, zeros_blk)
    h1, xr2 = _tc_combine(sp1, cntp, x, root1, bias1.reshape(1, _F), g1, bb1,
                          root2, bias2.reshape(1, _F))
    hs = _sc_gather(h1, src2d, _F)
    msg2 = _tc_msg(edge_attr, hs, nn2_w1.T, nn2_b1.reshape(1, _F),
                   nn2_w2.T, nn2_b2.reshape(1, _F * _F))
    sp2 = _sc_scatter(msg2, dst2d, zeros_blk)
    out_n = _tc_final(sp2, cntp, xr2, g2, bb2, lin1_w.T,
                      lin1_b.reshape(1, _OUT), lin2_w.T,
                      lin2_b.reshape(1, _OUT))
    return _sc_gather(out_n, src2d, _OUT)


# speedup vs baseline: 3.8398x; 2.6336x over previous
import functools
import math

import jax
import jax.numpy as jnp
from jax import lax
from jax.experimental import pallas as pl
from jax.experimental.pallas import tpu as pltpu
from jax.experimental.pallas import tpu_sc as plsc

_N = 10000
_E = 160000
_F = 16
_OUT = 8

_NC = 2
_NS = 16
_NW = _NC * _NS
_EPW = _E // _NW
_CH = 125
_NCH = _EPW // _CH
_IDXROWS = _E // _CH

_NPAD = 10112
_RPT = _NPAD // _NS

_BE = 3200
_GRID = _E // _BE

_SC_PARAMS = pltpu.CompilerParams(use_tc_tiling_on_sc=False)
_MESH_CACHE = []


def _mesh():
    if not _MESH_CACHE:
        _MESH_CACHE.append(plsc.VectorSubcoreMesh(
            core_axis_name="c", subcore_axis_name="s",
            num_cores=_NC, num_subcores=_NS))
    return _MESH_CACHE[0]


def _sc_gather_count(x, src2d, dst2d, ones_blk, zeros_blk):
    @functools.partial(
        pl.kernel,
        out_type=[
            jax.ShapeDtypeStruct((_E, _F), jnp.float32),
            jax.ShapeDtypeStruct((_NC, _NPAD, _F), jnp.float32),
        ],
        mesh=_mesh(),
        compiler_params=_SC_PARAMS,
        scratch_types=[
            pltpu.VMEM((_NCH, _CH), jnp.int32),
            pltpu.VMEM((_NCH, _CH), jnp.int32),
            pltpu.VMEM((_EPW, _F), jnp.float32),
            pltpu.VMEM((_CH, _F), jnp.float32),
            pltpu.SemaphoreType.DMA,
            pltpu.SemaphoreType.DMA,
            pltpu.VMEM_SHARED((_NPAD, _F), jnp.float32),
        ],
    )
    def k(x_hbm, src_hbm, dst_hbm, ones_hbm, zeros_hbm, xs_out, cnt_out,
          src_v, dst_v, rows_v, ones_v, gsem, ssem, acc):
        c = lax.axis_index("c")
        s = lax.axis_index("s")
        wid = c * _NS + s
        base = wid * _EPW
        crow = wid * _NCH
        pltpu.sync_copy(src_hbm.at[pl.ds(crow, _NCH)], src_v)
        pltpu.sync_copy(dst_hbm.at[pl.ds(crow, _NCH)], dst_v)
        pltpu.sync_copy(ones_hbm, ones_v)
        pltpu.sync_copy(zeros_hbm, acc.at[pl.ds(s * _RPT, _RPT)])
        gds = []
        for j in range(_NCH):
            gds.append(pltpu.async_copy(
                x_hbm.at[src_v.at[j]], rows_v.at[pl.ds(j * _CH, _CH)], gsem))
        plsc.subcore_barrier()
        sds = []
        for j in range(_NCH):
            sds.append(pltpu.async_copy(
                ones_v, acc.at[dst_v.at[j]], ssem, add=True))
        for d in gds:
            d.wait()
        pltpu.sync_copy(rows_v, xs_out.at[pl.ds(base, _EPW)])
        for d in sds:
            d.wait()
        plsc.subcore_barrier()
        pltpu.sync_copy(acc.at[pl.ds(s * _RPT, _RPT)],
                        cnt_out.at[c, pl.ds(s * _RPT, _RPT)])

    return k(x, src2d, dst2d, ones_blk, zeros_blk)


def _sc_scatter(msg, dst2d, zeros_blk):
    @functools.partial(
        pl.kernel,
        out_type=jax.ShapeDtypeStruct((_NC, _NPAD, _F), jnp.float32),
        mesh=_mesh(),
        compiler_params=_SC_PARAMS,
        scratch_types=[
            pltpu.VMEM((_NCH, _CH), jnp.int32),
            pltpu.VMEM((_EPW, _F), jnp.float32),
            pltpu.SemaphoreType.DMA,
            pltpu.VMEM_SHARED((_NPAD, _F), jnp.float32),
        ],
    )
    def k(msg_hbm, dst_hbm, zeros_hbm, part_out, dst_v, rows_v, ssem, acc):
        c = lax.axis_index("c")
        s = lax.axis_index("s")
        wid = c * _NS + s
        base = wid * _EPW
        crow = wid * _NCH
        pltpu.sync_copy(dst_hbm.at[pl.ds(crow, _NCH)], dst_v)
        pltpu.sync_copy(msg_hbm.at[pl.ds(base, _EPW)], rows_v)
        pltpu.sync_copy(zeros_hbm, acc.at[pl.ds(s * _RPT, _RPT)])
        plsc.subcore_barrier()
        sds = []
        for j in range(_NCH):
            sds.append(pltpu.async_copy(
                rows_v.at[pl.ds(j * _CH, _CH)], acc.at[dst_v.at[j]], ssem,
                add=True))
        for d in sds:
            d.wait()
        plsc.subcore_barrier()
        pltpu.sync_copy(acc.at[pl.ds(s * _RPT, _RPT)],
                        part_out.at[c, pl.ds(s * _RPT, _RPT)])

    return k(msg, dst2d, zeros_blk)


def _sc_gather(table, idx2d, width):
    @functools.partial(
        pl.kernel,
        out_type=jax.ShapeDtypeStruct((_E, width), jnp.float32),
        mesh=_mesh(),
        compiler_params=_SC_PARAMS,
        scratch_types=[
            pltpu.VMEM((_NCH, _CH), jnp.int32),
            pltpu.VMEM((_EPW, width), jnp.float32),
            pltpu.SemaphoreType.DMA,
        ],
    )
    def k(tab_hbm, idx_hbm, out_hbm, idx_v, rows_v, gsem):
        c = lax.axis_index("c")
        s = lax.axis_index("s")
        wid = c * _NS + s
        base = wid * _EPW
        crow = wid * _NCH
        pltpu.sync_copy(idx_hbm.at[pl.ds(crow, _NCH)], idx_v)
        gds = []
        for j in range(_NCH):
            gds.append(pltpu.async_copy(
                tab_hbm.at[idx_v.at[j]], rows_v.at[pl.ds(j * _CH, _CH)], gsem))
        for d in gds:
            d.wait()
        pltpu.sync_copy(rows_v, out_hbm.at[pl.ds(base, _EPW)])

    return k(table, idx2d)


def _msg_consts(w2, b2):
    f = _F
    k_idx = jnp.tile(jnp.arange(f), (f,))
    o_idx = jnp.repeat(jnp.arange(f), f)
    s_mat = (jnp.arange(f)[:, None] == k_idx[None, :]).astype(jnp.float32)
    c_mat = (o_idx[:, None] == jnp.arange(f)[None, :]).astype(jnp.float32)
    u_mat = w2.reshape(f, f, f).transpose(0, 2, 1)[:, k_idx, o_idx]
    b2m = b2.reshape(f, f)
    return s_mat, u_mat, c_mat, b2m


def _tc_msg(ea, xs, w1t, b1r, s_mat, u_mat, c_mat, b2m):
    def body(ea_ref, xs_ref, w1t_ref, b1_ref, s_ref, u_ref, c_ref, b2m_ref,
             out_ref):
        he = jnp.maximum(
            jnp.dot(ea_ref[...], w1t_ref[...],
                    preferred_element_type=jnp.float32) + b1_ref[...], 0.0)
        th = jnp.dot(he, s_ref[...], preferred_element_type=jnp.float32)
        g = jnp.dot(xs_ref[...], u_ref[...], preferred_element_type=jnp.float32)
        prod = th * g
        out_ref[...] = (
            jnp.dot(prod, c_ref[...], preferred_element_type=jnp.float32)
            + jnp.dot(xs_ref[...], b2m_ref[...],
                      preferred_element_type=jnp.float32))

    ff = _F * _F
    return pl.pallas_call(
        body,
        grid=(_GRID,),
        in_specs=[
            pl.BlockSpec((_BE, _F), lambda i: (i, 0)),
            pl.BlockSpec((_BE, _F), lambda i: (i, 0)),
            pl.BlockSpec((_F, _F), lambda i: (0, 0)),
            pl.BlockSpec((1, _F), lambda i: (0, 0)),
            pl.BlockSpec((_F, ff), lambda i: (0, 0)),
            pl.BlockSpec((_F, ff), lambda i: (0, 0)),
            pl.BlockSpec((ff, _F), lambda i: (0, 0)),
            pl.BlockSpec((_F, _F), lambda i: (0, 0)),
        ],
        out_specs=pl.BlockSpec((_BE, _F), lambda i: (i, 0)),
        out_shape=jax.ShapeDtypeStruct((_E, _F), jnp.float32),
    )(ea, xs, w1t, b1r, s_mat, u_mat, c_mat, b2m)


def _tc_combine(sp, cp, x, root1, bi1, g1, bb1, root2, bi2):
    def body(sp_ref, cp_ref, x_ref, r1_ref, bi1_ref, g_ref, bb_ref,
             r2_ref, bi2_ref, h1_ref, xr2_ref):
        ssum = sp_ref[0, :_N, :] + sp_ref[1, :_N, :]
        csum = cp_ref[0, :_N, :] + cp_ref[1, :_N, :]
        agg = ssum / jnp.maximum(csum[:, 0:1], 1.0)
        pre = agg + jnp.dot(x_ref[...], r1_ref[...],
                            preferred_element_type=jnp.float32) + bi1_ref[...]
        h1 = jnp.maximum(pre * g_ref[...] + bb_ref[...], 0.0)
        h1_ref[...] = h1
        xr2_ref[...] = jnp.dot(h1, r2_ref[...],
                               preferred_element_type=jnp.float32) + bi2_ref[...]

    return pl.pallas_call(
        body,
        out_shape=[
            jax.ShapeDtypeStruct((_N, _F), jnp.float32),
            jax.ShapeDtypeStruct((_N, _F), jnp.float32),
        ],
    )(sp, cp, x, root1, bi1, g1, bb1, root2, bi2)


def _tc_final(sp2, cp, xr2, g2, bb2, l1t, l1b, l2t, l2b):
    def body(sp_ref, cp_ref, xr2_ref, g_ref, bb_ref, l1t_ref, l1b_ref,
             l2t_ref, l2b_ref, out_ref):
        ssum = sp_ref[0, :_N, :] + sp_ref[1, :_N, :]
        csum = cp_ref[0, :_N, :] + cp_ref[1, :_N, :]
        agg = ssum / jnp.maximum(csum[:, 0:1], 1.0)
        pre = agg + xr2_ref[...]
        h2 = jnp.maximum(pre * g_ref[...] + bb_ref[...], 0.0)
        h3 = jnp.maximum(
            jnp.dot(h2, l1t_ref[...],
                    preferred_element_type=jnp.float32) + l1b_ref[...], 0.0)
        out_ref[...] = jnp.dot(h3, l2t_ref[...],
                               preferred_element_type=jnp.float32) + l2b_ref[...]

    return pl.pallas_call(
        body,
        out_shape=jax.ShapeDtypeStruct((_N, _OUT), jnp.float32),
    )(sp2, cp, xr2, g2, bb2, l1t, l1b, l2t, l2b)


def kernel(x, edge_index, edge_attr, nn1_w1, nn1_b1, nn1_w2, nn1_b2, root1,
           bias1, bn1_g, bn1_b, nn2_w1, nn2_b1, nn2_w2, nn2_b2, root2, bias2,
           bn2_g, bn2_b, lin1_w, lin1_b, lin2_w, lin2_b):
    src2d = edge_index[0].reshape(_IDXROWS, _CH)
    dst2d = edge_index[1].reshape(_IDXROWS, _CH)
    ones_blk = jnp.ones((_CH, _F), jnp.float32)
    zeros_blk = jnp.zeros((_RPT, _F), jnp.float32)

    bn_scale = 1.0 / math.sqrt(1.0 + 1e-5)
    g1 = (bn1_g * bn_scale).reshape(1, _F)
    bb1 = bn1_b.reshape(1, _F)
    g2 = (bn2_g * bn_scale).reshape(1, _F)
    bb2 = bn2_b.reshape(1, _F)

    s1, u1, c1, b2m1 = _msg_consts(nn1_w2, nn1_b2)
    s2, u2, c2, b2m2 = _msg_consts(nn2_w2, nn2_b2)

    xs, cntp = _sc_gather_count(x, src2d, dst2d, ones_blk, zeros_blk)
    msg1 = _tc_msg(edge_attr, xs, nn1_w1.T, nn1_b1.reshape(1, _F),
                   s1, u1, c1, b2m1)
    sp1 = _sc_scatter(msg1, dst2d, zeros_blk)
    h1, xr2 = _tc_combine(sp1, cntp, x, root1, bias1.reshape(1, _F), g1, bb1,
                          root2, bias2.reshape(1, _F))
    hs = _sc_gather(h1, src2d, _F)
    msg2 = _tc_msg(edge_attr, hs, nn2_w1.T, nn2_b1.reshape(1, _F),
                   s2, u2, c2, b2m2)
    sp2 = _sc_scatter(msg2, dst2d, zeros_blk)
    out_n = _tc_final(sp2, cntp, xr2, g2, bb2, lin1_w.T,
                      lin1_b.reshape(1, _OUT), lin2_w.T,
                      lin2_b.reshape(1, _OUT))
    return _sc_gather(out_n, src2d, _OUT)
